# Initial kernel scaffold; baseline (speedup 1.0000x reference)
import jax
import jax.numpy as jnp
from jax.experimental import pallas as pl


def kernel(x, fc_weight, bias, offsets):
    raise NotImplementedError("write your pallas kernel here")


# speedup vs baseline: 1.1723x; 1.1723x over previous
import functools

import jax
import jax.numpy as jnp
from jax import lax
from jax.experimental import pallas as pl
from jax.experimental.pallas import tpu as pltpu
from jax.experimental.pallas import tpu_sc as plsc

NUM_CORES = 2
NUM_SUBCORES = 16
NUM_WORKERS = NUM_CORES * NUM_SUBCORES
LANES = 16
CHUNK = 128
FIRE = 8


def _body(n_idx, n_chunks, b_per_w, num_fields,
          idx_hbm, tbl_hbm, bias_hbm, out_hbm,
          idx_v, gath_v, out_v, bias_v, sem):
    wid = lax.axis_index("s") * NUM_CORES + lax.axis_index("c")

    pltpu.sync_copy(idx_hbm.at[wid], idx_v)
    pltpu.sync_copy(bias_hbm, bias_v)

    def gather_group(g, _):
        base = g * FIRE
        copies = []
        for i in range(FIRE):
            c = base + i
            copies.append(pltpu.async_copy(
                tbl_hbm.at[idx_v.at[c]],
                gath_v.at[pl.ds(c * CHUNK, CHUNK)],
                sem))
        for cp in copies:
            cp.wait()
        return 0
    lax.fori_loop(0, n_chunks // FIRE, gather_group, 0)

    bias_vec = bias_v[...]

    def reduce_block(jb, _):
        base = jb * LANES
        acc = bias_vec
        for f in range(num_fields):
            acc = acc + gath_v[pl.ds(f * b_per_w + base, LANES)]
        out_v[pl.ds(base, LANES)] = acc
        return 0
    lax.fori_loop(0, b_per_w // LANES, reduce_block, 0)

    pltpu.sync_copy(out_v, out_hbm.at[pl.ds(wid * b_per_w, b_per_w)])


def kernel(x, fc_weight, bias, offsets):
    batch, num_fields = x.shape
    total = fc_weight.shape[0]
    b_per_w = batch // NUM_WORKERS
    n_idx = b_per_w * num_fields
    n_chunks = n_idx // CHUNK

    idx = (x.astype(jnp.int32) + offsets.astype(jnp.int32)[None, :])
    idx = idx.reshape(NUM_WORKERS, b_per_w, num_fields)
    idx = jnp.transpose(idx, (0, 2, 1)).reshape(NUM_WORKERS, n_chunks, CHUNK)
    tbl = fc_weight.reshape(total)
    bias16 = jnp.broadcast_to(bias.astype(jnp.float32), (LANES,))

    mesh = plsc.VectorSubcoreMesh(core_axis_name="c", subcore_axis_name="s",
                                  num_cores=NUM_CORES,
                                  num_subcores=NUM_SUBCORES)
    k = pl.kernel(
        functools.partial(_body, n_idx, n_chunks, b_per_w, num_fields),
        out_type=jax.ShapeDtypeStruct((batch,), jnp.float32),
        mesh=mesh,
        scratch_types=[
            pltpu.VMEM((n_chunks, CHUNK), jnp.int32),
            pltpu.VMEM((n_idx,), jnp.float32),
            pltpu.VMEM((b_per_w,), jnp.float32),
            pltpu.VMEM((LANES,), jnp.float32),
            pltpu.SemaphoreType.DMA,
        ],
    )
    out = k(idx, tbl, bias16)
    return out.reshape(batch, 1)


# speedup vs baseline: 1.1891x; 1.0143x over previous
import functools

import jax
import jax.numpy as jnp
from jax import lax
from jax.experimental import pallas as pl
from jax.experimental.pallas import tpu as pltpu
from jax.experimental.pallas import tpu_sc as plsc

NUM_CORES = 2
NUM_SUBCORES = 16
NUM_WORKERS = NUM_CORES * NUM_SUBCORES
LANES = 16
CHUNK = 128
FIRE = 8


def _body(n_idx, n_chunks, b_per_w, num_fields,
          idx_hbm, tbl_hbm, bias_hbm, out_hbm,
          idx_v, gath_v, out_v, bias_v, sem):
    wid = lax.axis_index("s") * NUM_CORES + lax.axis_index("c")

    pltpu.sync_copy(idx_hbm.at[wid], idx_v)
    pltpu.sync_copy(bias_hbm, bias_v)

    tbl_sq = tbl_hbm.at[0]

    def gather_group(g, _):
        base = g * FIRE
        copies = []
        for i in range(FIRE):
            c = base + i
            copies.append(pltpu.async_copy(
                tbl_sq.at[idx_v.at[c]],
                gath_v.at[pl.ds(c * CHUNK, CHUNK)],
                sem))
        for cp in copies:
            cp.wait()
        return 0
    lax.fori_loop(0, n_chunks // FIRE, gather_group, 0)

    bias_vec = bias_v[...]

    def reduce_block(jb, _):
        base = jb * LANES
        acc = bias_vec
        for f in range(num_fields):
            acc = acc + gath_v[pl.ds(f * b_per_w + base, LANES)]
        out_v[pl.ds(base, LANES)] = acc
        return 0
    lax.fori_loop(0, b_per_w // LANES, reduce_block, 0)

    pltpu.sync_copy(out_v, out_hbm.at[pl.ds(wid * b_per_w, b_per_w)])


def kernel(x, fc_weight, bias, offsets):
    batch, num_fields = x.shape
    total = fc_weight.shape[0]
    b_per_w = batch // NUM_WORKERS
    n_idx = b_per_w * num_fields
    n_chunks = n_idx // CHUNK

    idx = (x.astype(jnp.int32) + offsets.astype(jnp.int32)[None, :])
    idx = idx.reshape(NUM_WORKERS, b_per_w, num_fields)
    idx = jnp.transpose(idx, (0, 2, 1)).reshape(NUM_WORKERS, n_chunks, CHUNK)
    bias16 = jnp.broadcast_to(bias.astype(jnp.float32), (LANES,))

    mesh = plsc.VectorSubcoreMesh(core_axis_name="c", subcore_axis_name="s",
                                  num_cores=NUM_CORES,
                                  num_subcores=NUM_SUBCORES)
    k = pl.kernel(
        functools.partial(_body, n_idx, n_chunks, b_per_w, num_fields),
        out_type=jax.ShapeDtypeStruct((batch,), jnp.float32),
        mesh=mesh,
        compiler_params=pltpu.CompilerParams(use_tc_tiling_on_sc=False),
        scratch_types=[
            pltpu.VMEM((n_chunks, CHUNK), jnp.int32),
            pltpu.VMEM((n_idx,), jnp.float32),
            pltpu.VMEM((b_per_w,), jnp.float32),
            pltpu.VMEM((LANES,), jnp.float32),
            pltpu.SemaphoreType.DMA,
        ],
    )
    out = k(idx, fc_weight.reshape(1, total), bias16)
    return out.reshape(batch, 1)


# speedup vs baseline: 2.9073x; 2.4449x over previous
import functools

import jax
import jax.numpy as jnp
from jax import lax
from jax.experimental import pallas as pl
from jax.experimental.pallas import tpu as pltpu
from jax.experimental.pallas import tpu_sc as plsc

NUM_CORES = 2
NUM_SUBCORES = 16
NUM_WORKERS = NUM_CORES * NUM_SUBCORES
LANES = 16
CHUNK = 128
FIRE = 8


def _body(n_idx, n_chunks, b_per_w, num_fields,
          idx_hbm, tbl_hbm, bias_hbm, out_hbm,
          idx_v, gath_v, out_v, bias_v, sem):
    wid = lax.axis_index("s") * NUM_CORES + lax.axis_index("c")

    pltpu.sync_copy(idx_hbm.at[wid], idx_v)
    pltpu.sync_copy(bias_hbm, bias_v)

    tbl_sq = tbl_hbm.at[0]

    def gather_group(g, _):
        base = g * FIRE
        copies = []
        for i in range(FIRE):
            c = base + i
            copies.append(pltpu.async_copy(
                tbl_sq.at[idx_v.at[c]],
                gath_v.at[pl.ds(c * CHUNK, CHUNK)],
                sem))
        for cp in copies:
            cp.wait()
        return 0
    lax.fori_loop(0, n_chunks // FIRE, gather_group, 0)

    bias_vec = bias_v[...]

    def reduce_block(jb, _):
        base = jb * LANES
        acc = bias_vec
        for f in range(num_fields):
            acc = acc + gath_v[pl.ds(f * b_per_w + base, LANES)]
        out_v[pl.ds(base, LANES)] = acc
        return 0
    lax.fori_loop(0, b_per_w // LANES, reduce_block, 0)

    pltpu.sync_copy(out_v, out_hbm.at[pl.ds(wid * b_per_w, b_per_w)])


def kernel(x, fc_weight, bias, offsets):
    batch, num_fields = x.shape
    total = fc_weight.shape[0]
    b_per_w = batch // NUM_WORKERS
    n_idx = b_per_w * num_fields
    n_chunks = n_idx // CHUNK

    idx = (x.astype(jnp.int32) + offsets.astype(jnp.int32)[None, :])
    idx = idx.reshape(NUM_WORKERS, b_per_w, num_fields)
    idx = jnp.transpose(idx, (0, 2, 1)).reshape(NUM_WORKERS, n_chunks, CHUNK)
    bias16 = jnp.broadcast_to(bias.astype(jnp.float32), (LANES,))

    mesh = plsc.VectorSubcoreMesh(core_axis_name="c", subcore_axis_name="s",
                                  num_cores=NUM_CORES,
                                  num_subcores=NUM_SUBCORES)
    k = pl.kernel(
        functools.partial(_body, n_idx, n_chunks, b_per_w, num_fields),
        out_type=jax.ShapeDtypeStruct((batch,), jnp.float32),
        mesh=mesh,
        compiler_params=pltpu.CompilerParams(use_tc_tiling_on_sc=False),
        scratch_types=[
            pltpu.VMEM((n_chunks, CHUNK), jnp.int32),
            pltpu.VMEM((n_idx,), jnp.float32),
            pltpu.VMEM((b_per_w,), jnp.float32),
            pltpu.VMEM((LANES,), jnp.float32),
            pltpu.SemaphoreType.DMA,
        ],
    )
    padded_total = (total + 1023) // 1024 * 1024
    tbl = jnp.transpose(fc_weight, (1, 0))
    if padded_total != total:
        tbl = jnp.pad(tbl, ((0, 0), (0, padded_total - total)))
    out = k(idx, tbl, bias16)
    return out.reshape(batch, 1)


# speedup vs baseline: 3.3121x; 1.1392x over previous
import functools

import jax
import jax.numpy as jnp
from jax import lax
from jax.experimental import pallas as pl
from jax.experimental.pallas import tpu as pltpu
from jax.experimental.pallas import tpu_sc as plsc

NUM_CORES = 2
NUM_SUBCORES = 16
NUM_WORKERS = NUM_CORES * NUM_SUBCORES
LANES = 16
CHUNK = 128
FIRE = 8


def _body(n_idx, n_chunks, b_per_w, num_fields,
          idx_hbm, tbl_hbm, bias_hbm, out_hbm,
          idx_v, gath_v, out_v, bias_v, sem):
    wid = lax.axis_index("s") * NUM_CORES + lax.axis_index("c")

    pltpu.sync_copy(idx_hbm.at[wid], idx_v)
    pltpu.sync_copy(bias_hbm, bias_v)

    tbl_sq = tbl_hbm.at[0]

    copies = []
    for c in range(n_chunks):
        copies.append(pltpu.async_copy(
            tbl_sq.at[idx_v.at[c]],
            gath_v.at[pl.ds(c * CHUNK, CHUNK)],
            sem))
    for cp in copies:
        cp.wait()

    bias_vec = bias_v[...]

    def reduce_block(jb, _):
        base = jb * LANES
        acc = bias_vec
        for f in range(num_fields):
            acc = acc + gath_v[pl.ds(f * b_per_w + base, LANES)]
        out_v[pl.ds(base, LANES)] = acc
        return 0
    lax.fori_loop(0, b_per_w // LANES, reduce_block, 0)

    pltpu.sync_copy(out_v, out_hbm.at[pl.ds(wid * b_per_w, b_per_w)])


def kernel(x, fc_weight, bias, offsets):
    batch, num_fields = x.shape
    total = fc_weight.shape[0]
    b_per_w = batch // NUM_WORKERS
    n_idx = b_per_w * num_fields
    n_chunks = n_idx // CHUNK

    idx = (x.astype(jnp.int32) + offsets.astype(jnp.int32)[None, :])
    idx = idx.reshape(NUM_WORKERS, b_per_w, num_fields)
    idx = jnp.transpose(idx, (0, 2, 1)).reshape(NUM_WORKERS, n_chunks, CHUNK)
    bias16 = jnp.broadcast_to(bias.astype(jnp.float32), (LANES,))

    mesh = plsc.VectorSubcoreMesh(core_axis_name="c", subcore_axis_name="s",
                                  num_cores=NUM_CORES,
                                  num_subcores=NUM_SUBCORES)
    k = pl.kernel(
        functools.partial(_body, n_idx, n_chunks, b_per_w, num_fields),
        out_type=jax.ShapeDtypeStruct((batch,), jnp.float32),
        mesh=mesh,
        compiler_params=pltpu.CompilerParams(use_tc_tiling_on_sc=False),
        scratch_types=[
            pltpu.VMEM((n_chunks, CHUNK), jnp.int32),
            pltpu.VMEM((n_idx,), jnp.float32),
            pltpu.VMEM((b_per_w,), jnp.float32),
            pltpu.VMEM((LANES,), jnp.float32),
            pltpu.SemaphoreType.DMA,
        ],
    )
    padded_total = (total + 1023) // 1024 * 1024
    tbl = jnp.transpose(fc_weight, (1, 0))
    if padded_total != total:
        tbl = jnp.pad(tbl, ((0, 0), (0, padded_total - total)))
    out = k(idx, tbl, bias16)
    return out.reshape(batch, 1)


# speedup vs baseline: 3.3530x; 1.0124x over previous
import functools

import jax
import jax.numpy as jnp
from jax import lax
from jax.experimental import pallas as pl
from jax.experimental.pallas import tpu as pltpu
from jax.experimental.pallas import tpu_sc as plsc

NUM_CORES = 2
NUM_SUBCORES = 16
NUM_WORKERS = NUM_CORES * NUM_SUBCORES
LANES = 16
CHUNK = 128
FIRE = 8


def _body(n_idx, n_chunks, b_per_w, num_fields,
          idx_hbm, tbl_hbm, bias_hbm, out_hbm,
          idx_v, gath_v, out_v, bias_v, sem):
    wid = lax.axis_index("s") * NUM_CORES + lax.axis_index("c")

    pltpu.sync_copy(idx_hbm.at[wid], idx_v)
    pltpu.sync_copy(bias_hbm, bias_v)

    tbl_sq = tbl_hbm.at[0]
    pltpu.async_copy(tbl_sq.at[idx_v], gath_v, sem).wait()

    bias_vec = bias_v[...]

    def reduce_block(jb, _):
        base = jb * LANES
        acc = bias_vec
        for f in range(num_fields):
            acc = acc + gath_v[pl.ds(f * b_per_w + base, LANES)]
        out_v[pl.ds(base, LANES)] = acc
        return 0
    lax.fori_loop(0, b_per_w // LANES, reduce_block, 0)

    pltpu.sync_copy(out_v, out_hbm.at[pl.ds(wid * b_per_w, b_per_w)])


def kernel(x, fc_weight, bias, offsets):
    batch, num_fields = x.shape
    total = fc_weight.shape[0]
    b_per_w = batch // NUM_WORKERS
    n_idx = b_per_w * num_fields
    n_chunks = n_idx // CHUNK

    idx = (x.astype(jnp.int32) + offsets.astype(jnp.int32)[None, :])
    idx = idx.reshape(NUM_WORKERS, b_per_w, num_fields)
    idx = jnp.transpose(idx, (0, 2, 1)).reshape(NUM_WORKERS, n_idx)
    bias16 = jnp.broadcast_to(bias.astype(jnp.float32), (LANES,))

    mesh = plsc.VectorSubcoreMesh(core_axis_name="c", subcore_axis_name="s",
                                  num_cores=NUM_CORES,
                                  num_subcores=NUM_SUBCORES)
    k = pl.kernel(
        functools.partial(_body, n_idx, n_chunks, b_per_w, num_fields),
        out_type=jax.ShapeDtypeStruct((batch,), jnp.float32),
        mesh=mesh,
        compiler_params=pltpu.CompilerParams(use_tc_tiling_on_sc=False),
        scratch_types=[
            pltpu.VMEM((n_idx,), jnp.int32),
            pltpu.VMEM((n_idx,), jnp.float32),
            pltpu.VMEM((b_per_w,), jnp.float32),
            pltpu.VMEM((LANES,), jnp.float32),
            pltpu.SemaphoreType.DMA,
        ],
    )
    padded_total = (total + 1023) // 1024 * 1024
    tbl = fc_weight
    if padded_total != total:
        tbl = jnp.pad(tbl, ((0, padded_total - total), (0, 0)))
    tbl = jnp.transpose(tbl, (1, 0))
    out = k(idx, tbl, bias16)
    return out.reshape(batch, 1)


# speedup vs baseline: 3.3642x; 1.0033x over previous
import functools

import jax
import jax.numpy as jnp
from jax import lax
from jax.experimental import pallas as pl
from jax.experimental.pallas import tpu as pltpu
from jax.experimental.pallas import tpu_sc as plsc

NUM_CORES = 2
NUM_SUBCORES = 16
NUM_WORKERS = NUM_CORES * NUM_SUBCORES
LANES = 16
N_STREAMS = 4


def _body(n_idx, b_per_w, num_fields,
          idx_hbm, tbl_hbm, bias_hbm, out_hbm,
          idx_v, gath_v, out_v, bias_v, *sems):
    wid = lax.axis_index("s") * NUM_CORES + lax.axis_index("c")
    n_str = len(sems)
    seg = n_idx // n_str
    rows = b_per_w // n_str

    pltpu.sync_copy(idx_hbm.at[wid], idx_v)
    pltpu.sync_copy(bias_hbm, bias_v)

    tbl_sq = tbl_hbm.at[0]
    copies = [
        pltpu.async_copy(
            tbl_sq.at[idx_v.at[pl.ds(q * seg, seg)]],
            gath_v.at[pl.ds(q * seg, seg)],
            sems[q])
        for q in range(n_str)
    ]

    bias_vec = bias_v[...]
    for q in range(n_str):
        copies[q].wait()
        gbase = q * seg

        def reduce_block(jb, _, gbase=gbase, obase=q * rows):
            base = jb * LANES
            acc = bias_vec
            for f in range(num_fields):
                acc = acc + gath_v[pl.ds(gbase + f * rows + base, LANES)]
            out_v[pl.ds(obase + base, LANES)] = acc
            return 0
        lax.fori_loop(0, rows // LANES, reduce_block, 0)

    pltpu.sync_copy(out_v, out_hbm.at[pl.ds(wid * b_per_w, b_per_w)])


def kernel(x, fc_weight, bias, offsets):
    batch, num_fields = x.shape
    total = fc_weight.shape[0]
    b_per_w = batch // NUM_WORKERS
    n_idx = b_per_w * num_fields

    rows = b_per_w // N_STREAMS
    idx = (x.astype(jnp.int32) + offsets.astype(jnp.int32)[None, :])
    idx = idx.reshape(NUM_WORKERS, N_STREAMS, rows, num_fields)
    idx = jnp.transpose(idx, (0, 1, 3, 2)).reshape(NUM_WORKERS, n_idx)
    bias16 = jnp.broadcast_to(bias.astype(jnp.float32), (LANES,))

    mesh = plsc.VectorSubcoreMesh(core_axis_name="c", subcore_axis_name="s",
                                  num_cores=NUM_CORES,
                                  num_subcores=NUM_SUBCORES)
    k = pl.kernel(
        functools.partial(_body, n_idx, b_per_w, num_fields),
        out_type=jax.ShapeDtypeStruct((batch,), jnp.float32),
        mesh=mesh,
        compiler_params=pltpu.CompilerParams(use_tc_tiling_on_sc=False),
        scratch_types=[
            pltpu.VMEM((n_idx,), jnp.int32),
            pltpu.VMEM((n_idx,), jnp.float32),
            pltpu.VMEM((b_per_w,), jnp.float32),
            pltpu.VMEM((LANES,), jnp.float32),
        ] + [pltpu.SemaphoreType.DMA] * N_STREAMS,
    )
    padded_total = (total + 1023) // 1024 * 1024
    tbl = fc_weight
    if padded_total != total:
        tbl = jnp.pad(tbl, ((0, padded_total - total), (0, 0)))
    tbl = jnp.transpose(tbl, (1, 0))
    out = k(idx, tbl, bias16)
    return out.reshape(batch, 1)
